# baseline (speedup 1.0000x reference)
import functools

import jax
import jax.numpy as jnp
from jax import lax
from jax.experimental import pallas as pl
from jax.experimental.pallas import tpu as pltpu
from jax.experimental.pallas import tpu_sc as plsc

_CUTOFF = 10.0
_HID = 128
_NG = 50
_NGP = 64
_R = 32
_CT = 128
_NP = 10240
_NB = _NP // _R
_P = _R * _CT


def _ssp(x):
    return jnp.maximum(x, 0.0) + jnp.log1p(jnp.exp(-jnp.abs(x))) - jnp.log(2.0)


def _sc_embed(emb, zp):
    info = plsc.get_sparse_core_info()
    nc, ns = info.num_cores, info.num_subcores
    nw = nc * ns
    b_per_w = _NP // nw
    d = emb.shape[1]
    mesh = plsc.VectorSubcoreMesh(core_axis_name="c", subcore_axis_name="s")

    @functools.partial(
        pl.kernel,
        mesh=mesh,
        out_type=jax.ShapeDtypeStruct((_NP, d), jnp.float32),
        scratch_types=[
            pltpu.VMEM((b_per_w,), jnp.int32),
            pltpu.VMEM((b_per_w, d), jnp.float32),
            pltpu.SemaphoreType.DMA,
        ],
    )
    def gather_kernel(table_hbm, idx_hbm, out_hbm, idx_v, rows_v, sem):
        wid = lax.axis_index("s") * nc + lax.axis_index("c")
        base = wid * b_per_w
        pltpu.sync_copy(idx_hbm.at[pl.ds(base, b_per_w)], idx_v)
        pltpu.async_copy(table_hbm.at[idx_v], rows_v, sem).wait()
        pltpu.sync_copy(rows_v, out_hbm.at[pl.ds(base, b_per_w)])

    return gather_kernel(emb, zp)


def _banded_body(tlo_ref, tcnt_ref, nodef_ref, nodet_ref, xj_ref, iw1_ref,
                 ib1_ref, iw2_ref, ib2_ref, selr_ref, selc_ref, out_ref):
    b = pl.program_id(0)
    tlo = tlo_ref[b]
    tcnt = tcnt_ref[b]
    r0 = b * _R

    f32 = jnp.float32
    step = _CUTOFF / (_NG - 1)
    coeff = -0.5 / (step * step)
    gi = lax.broadcasted_iota(jnp.int32, (1, _NGP), 1).astype(f32)
    off = jnp.where(gi < _NG, gi * step, 1e6)

    rowdat = nodef_ref[pl.ds(r0, _R), :]
    br = rowdat[:, 0:1]
    nr = rowdat[:, 1:2]
    prx = rowdat[:, 2:3]
    pry = rowdat[:, 3:4]
    prz = rowdat[:, 4:5]
    ridx = r0 + lax.broadcasted_iota(jnp.int32, (_R, 1), 0)

    iw1v = iw1_ref[...]
    ib1v = ib1_ref[...]
    iw2v = iw2_ref[...]
    ib2v = ib2_ref[...]

    sel_r = selr_ref[...]
    sel_c = selc_ref[...]

    def _flatten2(a2, b2):
        ab = jnp.concatenate([a2, b2], axis=1)
        hi = ab.astype(jnp.bfloat16).astype(f32)
        mid = (ab - hi).astype(jnp.bfloat16).astype(f32)
        lo = ab - hi - mid
        ex = (jnp.dot(sel_r, hi, preferred_element_type=f32)
              + jnp.dot(sel_r, mid, preferred_element_type=f32)
              + jnp.dot(sel_r, lo, preferred_element_type=f32))
        af = jnp.sum(ex[:, :_CT] * sel_c, axis=1, keepdims=True)
        bf = jnp.sum(ex[:, _CT:] * sel_c, axis=1, keepdims=True)
        return af, bf

    def tile_body(t, acc):
        tt = tlo + t
        c0 = tt * _CT
        colt = nodet_ref[pl.ds(tt, 1), :, :].reshape(8, _CT)
        xjc = xj_ref[pl.ds(c0, _CT), :]
        bc = colt[0:1, :]
        nc = colt[1:2, :]
        pcx = colt[2:3, :]
        pcy = colt[3:4, :]
        pcz = colt[4:5, :]
        cidx = c0 + lax.broadcasted_iota(jnp.int32, (1, _CT), 1)

        dot3 = prx * pcx + pry * pcy + prz * pcz
        d2 = (nr + nc) - 2.0 * dot3
        m = (d2 < _CUTOFF * _CUTOFF) & (br == bc) & (ridx != cidx)
        dx = prx - pcx
        dy = pry - pcy
        dz = prz - pcz
        ew = jnp.sqrt(dx * dx + dy * dy + dz * dz)
        cw = 0.5 * (jnp.cos(ew * (jnp.pi / _CUTOFF)) + 1.0)
        cm2 = jnp.where(m, cw, 0.0)

        ew_f, cm_f = _flatten2(ew, cm2)
        ea = jnp.exp(coeff * (ew_f - off) ** 2)
        t1 = _ssp(jnp.dot(ea, iw1v, preferred_element_type=f32) + ib1v)
        w = jnp.dot(t1, iw2v, preferred_element_type=f32) + ib2v
        v = w * cm_f
        v3 = v.reshape(_R, _CT, _HID)
        return acc + jnp.sum(v3 * xjc[None, :, :], axis=1)

    acc = lax.fori_loop(0, tcnt, tile_body, jnp.zeros((_R, _HID), f32))
    out_ref[...] = acc


def _banded(tlo, tcnt, nodef, nodet, xj, iw1p, ib1, iw2, ib2):
    grid_spec = pltpu.PrefetchScalarGridSpec(
        num_scalar_prefetch=2,
        grid=(_NB,),
        in_specs=[
            pl.BlockSpec((_NP, 8), lambda b, *_: (0, 0)),
            pl.BlockSpec((_NP // _CT, 8, _CT), lambda b, *_: (0, 0, 0)),
            pl.BlockSpec((_NP, _HID), lambda b, *_: (0, 0)),
            pl.BlockSpec((_NGP, _HID), lambda b, *_: (0, 0)),
            pl.BlockSpec((1, _HID), lambda b, *_: (0, 0)),
            pl.BlockSpec((_HID, _HID), lambda b, *_: (0, 0)),
            pl.BlockSpec((1, _HID), lambda b, *_: (0, 0)),
            pl.BlockSpec((_P, _R), lambda b, *_: (0, 0)),
            pl.BlockSpec((_P, _CT), lambda b, *_: (0, 0)),
        ],
        out_specs=pl.BlockSpec((_R, _HID), lambda b, *_: (b, 0)),
    )
    selr = (jnp.arange(_P, dtype=jnp.int32)[:, None] // _CT
            == jnp.arange(_R, dtype=jnp.int32)[None, :]).astype(jnp.float32)
    selc = (jnp.arange(_P, dtype=jnp.int32)[:, None] % _CT
            == jnp.arange(_CT, dtype=jnp.int32)[None, :]).astype(jnp.float32)
    return pl.pallas_call(
        _banded_body,
        grid_spec=grid_spec,
        out_shape=jax.ShapeDtypeStruct((_NP, _HID), jnp.float32),
    )(tlo, tcnt, nodef, nodet, xj, iw1p, ib1.reshape(1, _HID), iw2,
      ib2.reshape(1, _HID), selr, selc)


def _mm_body(x_ref, w_ref, o_ref):
    o_ref[...] = jnp.dot(x_ref[...], w_ref[...],
                         preferred_element_type=jnp.float32)


def _mm(x, w):
    m, k = x.shape
    n = w.shape[1]
    blk = min(1024, m)
    return pl.pallas_call(
        _mm_body,
        grid=(m // blk,),
        in_specs=[pl.BlockSpec((blk, k), lambda i: (i, 0)),
                  pl.BlockSpec((k, n), lambda i: (0, 0))],
        out_specs=pl.BlockSpec((blk, n), lambda i: (i, 0)),
        out_shape=jax.ShapeDtypeStruct((m, n), jnp.float32),
    )(x, w)


def _update_body(h_ref, agg_ref, cw2_ref, cb2_ref, lw_ref, lb_ref, cw1n_ref,
                 h_out, xj_out):
    t = _ssp(jnp.dot(agg_ref[...], cw2_ref[...],
                     preferred_element_type=jnp.float32) + cb2_ref[...])
    hn = h_ref[...] + jnp.dot(
        t, lw_ref[...], preferred_element_type=jnp.float32) + lb_ref[...]
    h_out[...] = hn
    xj_out[...] = jnp.dot(hn, cw1n_ref[...],
                          preferred_element_type=jnp.float32)


def _update(h, agg, cw2, cb2, lw, lb, cw1n):
    blk = min(1024, _NP)
    return pl.pallas_call(
        _update_body,
        grid=(_NP // blk,),
        in_specs=[pl.BlockSpec((blk, _HID), lambda i: (i, 0)),
                  pl.BlockSpec((blk, _HID), lambda i: (i, 0)),
                  pl.BlockSpec((_HID, _HID), lambda i: (0, 0)),
                  pl.BlockSpec((1, _HID), lambda i: (0, 0)),
                  pl.BlockSpec((_HID, _HID), lambda i: (0, 0)),
                  pl.BlockSpec((1, _HID), lambda i: (0, 0)),
                  pl.BlockSpec((_HID, _HID), lambda i: (0, 0))],
        out_specs=[pl.BlockSpec((blk, _HID), lambda i: (i, 0)),
                   pl.BlockSpec((blk, _HID), lambda i: (i, 0))],
        out_shape=[jax.ShapeDtypeStruct((_NP, _HID), jnp.float32),
                   jax.ShapeDtypeStruct((_NP, _HID), jnp.float32)],
    )(h, agg, cw2, cb2.reshape(1, _HID), lw, lb.reshape(1, _HID), cw1n)


def _lin1_body(h_ref, w_ref, b_ref, o_ref):
    o_ref[...] = jnp.dot(h_ref[...], w_ref[...],
                         preferred_element_type=jnp.float32) + b_ref[...]


def _lin1(h, w, b):
    blk = min(1024, _NP)
    n = w.shape[1]
    return pl.pallas_call(
        _lin1_body,
        grid=(_NP // blk,),
        in_specs=[pl.BlockSpec((blk, _HID), lambda i: (i, 0)),
                  pl.BlockSpec((_HID, n), lambda i: (0, 0)),
                  pl.BlockSpec((1, n), lambda i: (0, 0))],
        out_specs=pl.BlockSpec((blk, n), lambda i: (i, 0)),
        out_shape=jax.ShapeDtypeStruct((_NP, n), jnp.float32),
    )(h, w, b.reshape(1, n))


def _readout_body(p_ref, m1w_ref, m1b_ref, m2w_ref, m2b_ref, o_ref):
    t = jax.nn.relu(jnp.dot(p_ref[...], m1w_ref[...],
                            preferred_element_type=jnp.float32) + m1b_ref[...])
    o_ref[...] = jnp.dot(t, m2w_ref[...],
                         preferred_element_type=jnp.float32) + m2b_ref[...]


def _readout(pairp, m1w, m1b, m2w, m2b):
    mp = pairp.shape[0]
    blk = 512
    return pl.pallas_call(
        _readout_body,
        grid=(mp // blk,),
        in_specs=[pl.BlockSpec((blk, _HID), lambda i: (i, 0)),
                  pl.BlockSpec((_HID, _HID), lambda i: (0, 0)),
                  pl.BlockSpec((1, _HID), lambda i: (0, 0)),
                  pl.BlockSpec((_HID, 1), lambda i: (0, 0)),
                  pl.BlockSpec((1, 1), lambda i: (0, 0))],
        out_specs=pl.BlockSpec((blk, 1), lambda i: (i, 0)),
        out_shape=jax.ShapeDtypeStruct((mp, 1), jnp.float32),
    )(pairp, m1w, m1b.reshape(1, _HID), m2w, m2b.reshape(1, 1))


def kernel(z, batch, pos, edges, emb, iw1, ib1, iw2, ib2, cw1, cw2, cb2,
           lw, lb, lin1_w, lin1_b, m1w, m1b, m2w, m2b):
    n = pos.shape[0]
    flat = edges[0].reshape(-1)
    pos_s = jnp.take(pos, flat, axis=0).astype(jnp.float32)
    nrm = (pos_s * pos_s).sum(1)
    batch_i = batch.astype(jnp.int32)

    padn = _NP - n
    batch_f = jnp.pad(batch_i, (0, padn),
                      constant_values=2 ** 24 - 1).astype(jnp.float32)
    nrm_p = jnp.pad(nrm, (0, padn))
    pos_p = jnp.pad(pos_s, ((0, padn), (0, 0)))
    gidx = jnp.arange(_NP, dtype=jnp.float32)
    zeros = jnp.zeros((_NP,), jnp.float32)
    nodef = jnp.stack([batch_f, nrm_p, pos_p[:, 0], pos_p[:, 1],
                       pos_p[:, 2], gidx, zeros, zeros], axis=1)
    nodet = nodef.T.reshape(8, _NP // _CT, _CT).transpose(1, 0, 2)

    row0 = jnp.arange(_NB, dtype=jnp.int32) * _R
    rlast = jnp.minimum(row0 + _R - 1, n - 1)
    bfirst = batch_i[jnp.minimum(row0, n - 1)]
    cs = jnp.searchsorted(batch_i, bfirst, side="left").astype(jnp.int32)
    ce = jnp.searchsorted(batch_i, batch_i[rlast], side="right").astype(jnp.int32)
    tlo = cs // _CT
    thi = (ce + _CT - 1) // _CT
    tcnt = jnp.where(row0 < n, thi - tlo, 0).astype(jnp.int32)

    iw1p = jnp.pad(iw1, ((0, 0), (0, _NGP - _NG), (0, 0)))

    zp = jnp.pad(z.astype(jnp.int32), (0, padn))
    h = _sc_embed(emb.astype(jnp.float32), zp)

    xj = _mm(h, cw1[0])
    for i in range(6):
        agg = _banded(tlo, tcnt, nodef, nodet, xj, iw1p[i], ib1[i], iw2[i],
                      ib2[i])
        h, xj = _update(h, agg, cw2[i], cb2[i], lw[i], lb[i],
                        cw1[(i + 1) % 6])

    ne = _lin1(h, lin1_w, lin1_b)
    pair = ne[:n].reshape(n // 2, 2 * ne.shape[1])
    mp = 5120
    pairp = jnp.pad(pair, ((0, mp - n // 2), (0, 0)))
    outp = _readout(pairp, m1w, m1b, m2w, m2b)
    return outp[: n // 2, 0]


# speedup vs baseline: 1.5620x; 1.5620x over previous
import functools

import jax
import jax.numpy as jnp
from jax import lax
from jax.experimental import pallas as pl
from jax.experimental.pallas import tpu as pltpu
from jax.experimental.pallas import tpu_sc as plsc

_CUTOFF = 10.0
_HID = 128
_NG = 50
_NGP = 64
_R = 32
_CT = 128
_NP = 10240
_NB = _NP // _R
_P = _R * _CT


def _ssp(x):
    return jnp.maximum(x, 0.0) + jnp.log1p(jnp.exp(-jnp.abs(x))) - jnp.log(2.0)


def _sc_embed(emb, zp):
    info = plsc.get_sparse_core_info()
    nc, ns = info.num_cores, info.num_subcores
    nw = nc * ns
    b_per_w = _NP // nw
    d = emb.shape[1]
    mesh = plsc.VectorSubcoreMesh(core_axis_name="c", subcore_axis_name="s")

    @functools.partial(
        pl.kernel,
        mesh=mesh,
        out_type=jax.ShapeDtypeStruct((_NP, d), jnp.float32),
        scratch_types=[
            pltpu.VMEM((b_per_w,), jnp.int32),
            pltpu.VMEM((b_per_w, d), jnp.float32),
            pltpu.SemaphoreType.DMA,
        ],
    )
    def gather_kernel(table_hbm, idx_hbm, out_hbm, idx_v, rows_v, sem):
        wid = lax.axis_index("s") * nc + lax.axis_index("c")
        base = wid * b_per_w
        pltpu.sync_copy(idx_hbm.at[pl.ds(base, b_per_w)], idx_v)
        pltpu.async_copy(table_hbm.at[idx_v], rows_v, sem).wait()
        pltpu.sync_copy(rows_v, out_hbm.at[pl.ds(base, b_per_w)])

    return gather_kernel(emb, zp)


def _banded_body(tlo_ref, tcnt_ref, nodef_ref, nodet_ref, xj_ref, iw1_ref,
                 ib1_ref, iw2_ref, ib2_ref, selc_ref, out_ref):
    b = pl.program_id(0)
    tlo = tlo_ref[b]
    tcnt = tcnt_ref[b]
    r0 = b * _R

    f32 = jnp.float32
    step = _CUTOFF / (_NG - 1)
    coeff = -0.5 / (step * step)
    gi = lax.broadcasted_iota(jnp.int32, (1, _NGP), 1).astype(f32)
    off = jnp.where(gi < _NG, gi * step, 1e6)

    rowdat = nodef_ref[pl.ds(r0, _R), :]
    br = rowdat[:, 0:1]
    nr = rowdat[:, 1:2]
    prx = rowdat[:, 2:3]
    pry = rowdat[:, 3:4]
    prz = rowdat[:, 4:5]
    ridx = r0 + lax.broadcasted_iota(jnp.int32, (_R, 1), 0)

    iw1v = iw1_ref[...]
    ib1v = ib1_ref[...]
    iw2v = iw2_ref[...]
    ib2v = ib2_ref[...]

    sel_c = selc_ref[...]

    def _flatten2(a2, b2):
        ab = jnp.concatenate([a2, b2], axis=1)
        ex = jnp.repeat(ab, _CT, axis=0)
        af = jnp.sum(ex[:, :_CT] * sel_c, axis=1, keepdims=True)
        bf = jnp.sum(ex[:, _CT:] * sel_c, axis=1, keepdims=True)
        return af, bf

    def tile_body(t, acc):
        tt = tlo + t
        c0 = tt * _CT
        colt = nodet_ref[pl.ds(tt, 1), :, :].reshape(8, _CT)
        xjc = xj_ref[pl.ds(c0, _CT), :]
        bc = colt[0:1, :]
        nc = colt[1:2, :]
        pcx = colt[2:3, :]
        pcy = colt[3:4, :]
        pcz = colt[4:5, :]
        cidx = c0 + lax.broadcasted_iota(jnp.int32, (1, _CT), 1)

        dot3 = prx * pcx + pry * pcy + prz * pcz
        d2 = (nr + nc) - 2.0 * dot3
        m = (d2 < _CUTOFF * _CUTOFF) & (br == bc) & (ridx != cidx)
        dx = prx - pcx
        dy = pry - pcy
        dz = prz - pcz
        ew = jnp.sqrt(dx * dx + dy * dy + dz * dz)
        cw = 0.5 * (jnp.cos(ew * (jnp.pi / _CUTOFF)) + 1.0)
        cm2 = jnp.where(m, cw, 0.0)

        ew_f, cm_f = _flatten2(ew, cm2)
        ea = jnp.exp(coeff * (ew_f - off) ** 2)
        t1 = _ssp(jnp.dot(ea, iw1v, preferred_element_type=f32) + ib1v)
        w = jnp.dot(t1, iw2v, preferred_element_type=f32) + ib2v
        v = w * cm_f
        v3 = v.reshape(_R, _CT, _HID)
        return acc + jnp.sum(v3 * xjc[None, :, :], axis=1)

    acc = lax.fori_loop(0, tcnt, tile_body, jnp.zeros((_R, _HID), f32))
    out_ref[...] = acc


def _banded(tlo, tcnt, nodef, nodet, xj, iw1p, ib1, iw2, ib2):
    grid_spec = pltpu.PrefetchScalarGridSpec(
        num_scalar_prefetch=2,
        grid=(_NB,),
        in_specs=[
            pl.BlockSpec((_NP, 8), lambda b, *_: (0, 0)),
            pl.BlockSpec((_NP // _CT, 8, _CT), lambda b, *_: (0, 0, 0)),
            pl.BlockSpec((_NP, _HID), lambda b, *_: (0, 0)),
            pl.BlockSpec((_NGP, _HID), lambda b, *_: (0, 0)),
            pl.BlockSpec((1, _HID), lambda b, *_: (0, 0)),
            pl.BlockSpec((_HID, _HID), lambda b, *_: (0, 0)),
            pl.BlockSpec((1, _HID), lambda b, *_: (0, 0)),
            pl.BlockSpec((_P, _CT), lambda b, *_: (0, 0)),
        ],
        out_specs=pl.BlockSpec((_R, _HID), lambda b, *_: (b, 0)),
    )
    selc = (jnp.arange(_P, dtype=jnp.int32)[:, None] % _CT
            == jnp.arange(_CT, dtype=jnp.int32)[None, :]).astype(jnp.float32)
    return pl.pallas_call(
        _banded_body,
        grid_spec=grid_spec,
        out_shape=jax.ShapeDtypeStruct((_NP, _HID), jnp.float32),
    )(tlo, tcnt, nodef, nodet, xj, iw1p, ib1.reshape(1, _HID), iw2,
      ib2.reshape(1, _HID), selc)


def _mm_body(x_ref, w_ref, o_ref):
    o_ref[...] = jnp.dot(x_ref[...], w_ref[...],
                         preferred_element_type=jnp.float32)


def _mm(x, w):
    m, k = x.shape
    n = w.shape[1]
    blk = min(1024, m)
    return pl.pallas_call(
        _mm_body,
        grid=(m // blk,),
        in_specs=[pl.BlockSpec((blk, k), lambda i: (i, 0)),
                  pl.BlockSpec((k, n), lambda i: (0, 0))],
        out_specs=pl.BlockSpec((blk, n), lambda i: (i, 0)),
        out_shape=jax.ShapeDtypeStruct((m, n), jnp.float32),
    )(x, w)


def _update_body(h_ref, agg_ref, cw2_ref, cb2_ref, lw_ref, lb_ref, cw1n_ref,
                 h_out, xj_out):
    t = _ssp(jnp.dot(agg_ref[...], cw2_ref[...],
                     preferred_element_type=jnp.float32) + cb2_ref[...])
    hn = h_ref[...] + jnp.dot(
        t, lw_ref[...], preferred_element_type=jnp.float32) + lb_ref[...]
    h_out[...] = hn
    xj_out[...] = jnp.dot(hn, cw1n_ref[...],
                          preferred_element_type=jnp.float32)


def _update(h, agg, cw2, cb2, lw, lb, cw1n):
    blk = min(1024, _NP)
    return pl.pallas_call(
        _update_body,
        grid=(_NP // blk,),
        in_specs=[pl.BlockSpec((blk, _HID), lambda i: (i, 0)),
                  pl.BlockSpec((blk, _HID), lambda i: (i, 0)),
                  pl.BlockSpec((_HID, _HID), lambda i: (0, 0)),
                  pl.BlockSpec((1, _HID), lambda i: (0, 0)),
                  pl.BlockSpec((_HID, _HID), lambda i: (0, 0)),
                  pl.BlockSpec((1, _HID), lambda i: (0, 0)),
                  pl.BlockSpec((_HID, _HID), lambda i: (0, 0))],
        out_specs=[pl.BlockSpec((blk, _HID), lambda i: (i, 0)),
                   pl.BlockSpec((blk, _HID), lambda i: (i, 0))],
        out_shape=[jax.ShapeDtypeStruct((_NP, _HID), jnp.float32),
                   jax.ShapeDtypeStruct((_NP, _HID), jnp.float32)],
    )(h, agg, cw2, cb2.reshape(1, _HID), lw, lb.reshape(1, _HID), cw1n)


def _lin1_body(h_ref, w_ref, b_ref, o_ref):
    o_ref[...] = jnp.dot(h_ref[...], w_ref[...],
                         preferred_element_type=jnp.float32) + b_ref[...]


def _lin1(h, w, b):
    blk = min(1024, _NP)
    n = w.shape[1]
    return pl.pallas_call(
        _lin1_body,
        grid=(_NP // blk,),
        in_specs=[pl.BlockSpec((blk, _HID), lambda i: (i, 0)),
                  pl.BlockSpec((_HID, n), lambda i: (0, 0)),
                  pl.BlockSpec((1, n), lambda i: (0, 0))],
        out_specs=pl.BlockSpec((blk, n), lambda i: (i, 0)),
        out_shape=jax.ShapeDtypeStruct((_NP, n), jnp.float32),
    )(h, w, b.reshape(1, n))


def _readout_body(p_ref, m1w_ref, m1b_ref, m2w_ref, m2b_ref, o_ref):
    t = jax.nn.relu(jnp.dot(p_ref[...], m1w_ref[...],
                            preferred_element_type=jnp.float32) + m1b_ref[...])
    o_ref[...] = jnp.dot(t, m2w_ref[...],
                         preferred_element_type=jnp.float32) + m2b_ref[...]


def _readout(pairp, m1w, m1b, m2w, m2b):
    mp = pairp.shape[0]
    blk = 512
    return pl.pallas_call(
        _readout_body,
        grid=(mp // blk,),
        in_specs=[pl.BlockSpec((blk, _HID), lambda i: (i, 0)),
                  pl.BlockSpec((_HID, _HID), lambda i: (0, 0)),
                  pl.BlockSpec((1, _HID), lambda i: (0, 0)),
                  pl.BlockSpec((_HID, 1), lambda i: (0, 0)),
                  pl.BlockSpec((1, 1), lambda i: (0, 0))],
        out_specs=pl.BlockSpec((blk, 1), lambda i: (i, 0)),
        out_shape=jax.ShapeDtypeStruct((mp, 1), jnp.float32),
    )(pairp, m1w, m1b.reshape(1, _HID), m2w, m2b.reshape(1, 1))


def kernel(z, batch, pos, edges, emb, iw1, ib1, iw2, ib2, cw1, cw2, cb2,
           lw, lb, lin1_w, lin1_b, m1w, m1b, m2w, m2b):
    n = pos.shape[0]
    flat = edges[0].reshape(-1)
    pos_s = jnp.take(pos, flat, axis=0).astype(jnp.float32)
    nrm = (pos_s * pos_s).sum(1)
    batch_i = batch.astype(jnp.int32)

    padn = _NP - n
    batch_f = jnp.pad(batch_i, (0, padn),
                      constant_values=2 ** 24 - 1).astype(jnp.float32)
    nrm_p = jnp.pad(nrm, (0, padn))
    pos_p = jnp.pad(pos_s, ((0, padn), (0, 0)))
    gidx = jnp.arange(_NP, dtype=jnp.float32)
    zeros = jnp.zeros((_NP,), jnp.float32)
    nodef = jnp.stack([batch_f, nrm_p, pos_p[:, 0], pos_p[:, 1],
                       pos_p[:, 2], gidx, zeros, zeros], axis=1)
    nodet = nodef.T.reshape(8, _NP // _CT, _CT).transpose(1, 0, 2)

    row0 = jnp.arange(_NB, dtype=jnp.int32) * _R
    rlast = jnp.minimum(row0 + _R - 1, n - 1)
    bfirst = batch_i[jnp.minimum(row0, n - 1)]
    cs = jnp.searchsorted(batch_i, bfirst, side="left").astype(jnp.int32)
    ce = jnp.searchsorted(batch_i, batch_i[rlast], side="right").astype(jnp.int32)
    tlo = cs // _CT
    thi = (ce + _CT - 1) // _CT
    tcnt = jnp.where(row0 < n, thi - tlo, 0).astype(jnp.int32)

    iw1p = jnp.pad(iw1, ((0, 0), (0, _NGP - _NG), (0, 0)))

    zp = jnp.pad(z.astype(jnp.int32), (0, padn))
    h = _sc_embed(emb.astype(jnp.float32), zp)

    xj = _mm(h, cw1[0])
    for i in range(6):
        agg = _banded(tlo, tcnt, nodef, nodet, xj, iw1p[i], ib1[i], iw2[i],
                      ib2[i])
        h, xj = _update(h, agg, cw2[i], cb2[i], lw[i], lb[i],
                        cw1[(i + 1) % 6])

    ne = _lin1(h, lin1_w, lin1_b)
    pair = ne[:n].reshape(n // 2, 2 * ne.shape[1])
    mp = 5120
    pairp = jnp.pad(pair, ((0, mp - n // 2), (0, 0)))
    outp = _readout(pairp, m1w, m1b, m2w, m2b)
    return outp[: n // 2, 0]


# speedup vs baseline: 2.0998x; 1.3443x over previous
import functools

import jax
import jax.numpy as jnp
from jax import lax
from jax.experimental import pallas as pl
from jax.experimental.pallas import tpu as pltpu
from jax.experimental.pallas import tpu_sc as plsc

_CUTOFF = 10.0
_HID = 128
_NG = 50
_NGP = 64
_R = 32
_CT = 128
_NP = 10240
_NB = _NP // _R
_P = _R * _CT


def _ssp(x):
    return jnp.log(jnp.exp(x) + 1.0) - jnp.log(2.0)


def _sc_embed(emb, zp):
    info = plsc.get_sparse_core_info()
    nc, ns = info.num_cores, info.num_subcores
    nw = nc * ns
    b_per_w = _NP // nw
    d = emb.shape[1]
    mesh = plsc.VectorSubcoreMesh(core_axis_name="c", subcore_axis_name="s")

    @functools.partial(
        pl.kernel,
        mesh=mesh,
        out_type=jax.ShapeDtypeStruct((_NP, d), jnp.float32),
        scratch_types=[
            pltpu.VMEM((b_per_w,), jnp.int32),
            pltpu.VMEM((b_per_w, d), jnp.float32),
            pltpu.SemaphoreType.DMA,
        ],
    )
    def gather_kernel(table_hbm, idx_hbm, out_hbm, idx_v, rows_v, sem):
        wid = lax.axis_index("s") * nc + lax.axis_index("c")
        base = wid * b_per_w
        pltpu.sync_copy(idx_hbm.at[pl.ds(base, b_per_w)], idx_v)
        pltpu.async_copy(table_hbm.at[idx_v], rows_v, sem).wait()
        pltpu.sync_copy(rows_v, out_hbm.at[pl.ds(base, b_per_w)])

    return gather_kernel(emb, zp)


def _banded_body(tlo_ref, tcnt_ref, nodef_ref, nodet_ref, xj_ref, iw1_ref,
                 ib1_ref, iw2_ref, ib2_ref, selc_ref, out_ref):
    b = pl.program_id(0)
    tlo = tlo_ref[b]
    tcnt = tcnt_ref[b]
    r0 = b * _R

    f32 = jnp.float32
    step = _CUTOFF / (_NG - 1)
    coeff = -0.5 / (step * step)
    gi = lax.broadcasted_iota(jnp.int32, (1, _NGP), 1).astype(f32)
    off = jnp.where(gi < _NG, gi * step, 1e6)

    rowdat = nodef_ref[pl.ds(r0, _R), :]
    br = rowdat[:, 0:1]
    nr = rowdat[:, 1:2]
    prx = rowdat[:, 2:3]
    pry = rowdat[:, 3:4]
    prz = rowdat[:, 4:5]
    ridx = r0 + lax.broadcasted_iota(jnp.int32, (_R, 1), 0)

    iw1v = iw1_ref[...]
    ib1v = ib1_ref[...]
    iw2v = iw2_ref[...]
    ib2v = ib2_ref[...]

    sel_c = selc_ref[...]

    def _flatten2(a2, b2):
        ab = jnp.concatenate([a2, b2], axis=1)
        ex = jnp.repeat(ab, _CT, axis=0)
        af = jnp.sum(ex[:, :_CT] * sel_c, axis=1, keepdims=True)
        bf = jnp.sum(ex[:, _CT:] * sel_c, axis=1, keepdims=True)
        return af, bf

    def tile_body(t, acc):
        tt = tlo + t
        c0 = tt * _CT
        colt = nodet_ref[pl.ds(tt * 8, 8), :]
        xjc = xj_ref[pl.ds(c0, _CT), :]
        bc = colt[0:1, :]
        nc = colt[1:2, :]
        pcx = colt[2:3, :]
        pcy = colt[3:4, :]
        pcz = colt[4:5, :]
        cidx = c0 + lax.broadcasted_iota(jnp.int32, (1, _CT), 1)

        dot3 = prx * pcx + pry * pcy + prz * pcz
        d2 = (nr + nc) - 2.0 * dot3
        m = (d2 < _CUTOFF * _CUTOFF) & (br == bc) & (ridx != cidx)
        dx = prx - pcx
        dy = pry - pcy
        dz = prz - pcz
        ew = jnp.sqrt(dx * dx + dy * dy + dz * dz)
        cw = 0.5 * (jnp.cos(ew * (jnp.pi / _CUTOFF)) + 1.0)
        cm2 = jnp.where(m, cw, 0.0)

        ew_f, cm_f = _flatten2(ew, cm2)
        ea = jnp.exp(coeff * (ew_f - off) ** 2)
        t1 = _ssp(jnp.dot(ea, iw1v, preferred_element_type=f32) + ib1v)
        w = jnp.dot(t1, iw2v, preferred_element_type=f32) + ib2v
        v = w * cm_f
        v3 = v.reshape(_R, _CT, _HID)
        return acc + jnp.sum(v3 * xjc[None, :, :], axis=1)

    acc = lax.fori_loop(0, tcnt, tile_body, jnp.zeros((_R, _HID), f32))
    out_ref[...] = acc


def _banded(tlo, tcnt, nodef, nodet, xj, iw1p, ib1, iw2, ib2):
    grid_spec = pltpu.PrefetchScalarGridSpec(
        num_scalar_prefetch=2,
        grid=(_NB,),
        in_specs=[
            pl.BlockSpec((_NP, 8), lambda b, *_: (0, 0)),
            pl.BlockSpec((_NP // _CT * 8, _CT), lambda b, *_: (0, 0)),
            pl.BlockSpec((_NP, _HID), lambda b, *_: (0, 0)),
            pl.BlockSpec((_NGP, _HID), lambda b, *_: (0, 0)),
            pl.BlockSpec((1, _HID), lambda b, *_: (0, 0)),
            pl.BlockSpec((_HID, _HID), lambda b, *_: (0, 0)),
            pl.BlockSpec((1, _HID), lambda b, *_: (0, 0)),
            pl.BlockSpec((_P, _CT), lambda b, *_: (0, 0)),
        ],
        out_specs=pl.BlockSpec((_R, _HID), lambda b, *_: (b, 0)),
    )
    selc = (jnp.arange(_P, dtype=jnp.int32)[:, None] % _CT
            == jnp.arange(_CT, dtype=jnp.int32)[None, :]).astype(jnp.float32)
    return pl.pallas_call(
        _banded_body,
        grid_spec=grid_spec,
        out_shape=jax.ShapeDtypeStruct((_NP, _HID), jnp.float32),
    )(tlo, tcnt, nodef, nodet, xj, iw1p, ib1.reshape(1, _HID), iw2,
      ib2.reshape(1, _HID), selc)


def _mm_body(x_ref, w_ref, o_ref):
    o_ref[...] = jnp.dot(x_ref[...], w_ref[...],
                         preferred_element_type=jnp.float32)


def _mm(x, w):
    m, k = x.shape
    n = w.shape[1]
    blk = min(1024, m)
    return pl.pallas_call(
        _mm_body,
        grid=(m // blk,),
        in_specs=[pl.BlockSpec((blk, k), lambda i: (i, 0)),
                  pl.BlockSpec((k, n), lambda i: (0, 0))],
        out_specs=pl.BlockSpec((blk, n), lambda i: (i, 0)),
        out_shape=jax.ShapeDtypeStruct((m, n), jnp.float32),
    )(x, w)


def _update_body(h_ref, agg_ref, cw2_ref, cb2_ref, lw_ref, lb_ref, cw1n_ref,
                 h_out, xj_out):
    t = _ssp(jnp.dot(agg_ref[...], cw2_ref[...],
                     preferred_element_type=jnp.float32) + cb2_ref[...])
    hn = h_ref[...] + jnp.dot(
        t, lw_ref[...], preferred_element_type=jnp.float32) + lb_ref[...]
    h_out[...] = hn
    xj_out[...] = jnp.dot(hn, cw1n_ref[...],
                          preferred_element_type=jnp.float32)


def _update(h, agg, cw2, cb2, lw, lb, cw1n):
    blk = min(1024, _NP)
    return pl.pallas_call(
        _update_body,
        grid=(_NP // blk,),
        in_specs=[pl.BlockSpec((blk, _HID), lambda i: (i, 0)),
                  pl.BlockSpec((blk, _HID), lambda i: (i, 0)),
                  pl.BlockSpec((_HID, _HID), lambda i: (0, 0)),
                  pl.BlockSpec((1, _HID), lambda i: (0, 0)),
                  pl.BlockSpec((_HID, _HID), lambda i: (0, 0)),
                  pl.BlockSpec((1, _HID), lambda i: (0, 0)),
                  pl.BlockSpec((_HID, _HID), lambda i: (0, 0))],
        out_specs=[pl.BlockSpec((blk, _HID), lambda i: (i, 0)),
                   pl.BlockSpec((blk, _HID), lambda i: (i, 0))],
        out_shape=[jax.ShapeDtypeStruct((_NP, _HID), jnp.float32),
                   jax.ShapeDtypeStruct((_NP, _HID), jnp.float32)],
    )(h, agg, cw2, cb2.reshape(1, _HID), lw, lb.reshape(1, _HID), cw1n)


def _lin1_body(h_ref, w_ref, b_ref, o_ref):
    o_ref[...] = jnp.dot(h_ref[...], w_ref[...],
                         preferred_element_type=jnp.float32) + b_ref[...]


def _lin1(h, w, b):
    blk = min(1024, _NP)
    n = w.shape[1]
    return pl.pallas_call(
        _lin1_body,
        grid=(_NP // blk,),
        in_specs=[pl.BlockSpec((blk, _HID), lambda i: (i, 0)),
                  pl.BlockSpec((_HID, n), lambda i: (0, 0)),
                  pl.BlockSpec((1, n), lambda i: (0, 0))],
        out_specs=pl.BlockSpec((blk, n), lambda i: (i, 0)),
        out_shape=jax.ShapeDtypeStruct((_NP, n), jnp.float32),
    )(h, w, b.reshape(1, n))


def _readout_body(p_ref, m1w_ref, m1b_ref, m2w_ref, m2b_ref, o_ref):
    t = jax.nn.relu(jnp.dot(p_ref[...], m1w_ref[...],
                            preferred_element_type=jnp.float32) + m1b_ref[...])
    o_ref[...] = jnp.dot(t, m2w_ref[...],
                         preferred_element_type=jnp.float32) + m2b_ref[...]


def _readout(pairp, m1w, m1b, m2w, m2b):
    mp = pairp.shape[0]
    blk = 512
    return pl.pallas_call(
        _readout_body,
        grid=(mp // blk,),
        in_specs=[pl.BlockSpec((blk, _HID), lambda i: (i, 0)),
                  pl.BlockSpec((_HID, _HID), lambda i: (0, 0)),
                  pl.BlockSpec((1, _HID), lambda i: (0, 0)),
                  pl.BlockSpec((_HID, 1), lambda i: (0, 0)),
                  pl.BlockSpec((1, 1), lambda i: (0, 0))],
        out_specs=pl.BlockSpec((blk, 1), lambda i: (i, 0)),
        out_shape=jax.ShapeDtypeStruct((mp, 1), jnp.float32),
    )(pairp, m1w, m1b.reshape(1, _HID), m2w, m2b.reshape(1, 1))


def kernel(z, batch, pos, edges, emb, iw1, ib1, iw2, ib2, cw1, cw2, cb2,
           lw, lb, lin1_w, lin1_b, m1w, m1b, m2w, m2b):
    n = pos.shape[0]
    flat = edges[0].reshape(-1)
    pos_s = jnp.take(pos, flat, axis=0).astype(jnp.float32)
    nrm = (pos_s * pos_s).sum(1)
    batch_i = batch.astype(jnp.int32)

    padn = _NP - n
    batch_f = jnp.pad(batch_i, (0, padn),
                      constant_values=2 ** 24 - 1).astype(jnp.float32)
    nrm_p = jnp.pad(nrm, (0, padn))
    pos_p = jnp.pad(pos_s, ((0, padn), (0, 0)))
    gidx = jnp.arange(_NP, dtype=jnp.float32)
    zeros = jnp.zeros((_NP,), jnp.float32)
    nodef = jnp.stack([batch_f, nrm_p, pos_p[:, 0], pos_p[:, 1],
                       pos_p[:, 2], gidx, zeros, zeros], axis=1)
    nodet = (nodef.T.reshape(8, _NP // _CT, _CT).transpose(1, 0, 2)
             .reshape(_NP // _CT * 8, _CT))

    row0 = jnp.arange(_NB, dtype=jnp.int32) * _R
    rlast = jnp.minimum(row0 + _R - 1, n - 1)
    bfirst = batch_i[jnp.minimum(row0, n - 1)]
    cs = jnp.searchsorted(batch_i, bfirst, side="left").astype(jnp.int32)
    ce = jnp.searchsorted(batch_i, batch_i[rlast], side="right").astype(jnp.int32)
    tlo = cs // _CT
    thi = (ce + _CT - 1) // _CT
    tcnt = jnp.where(row0 < n, thi - tlo, 0).astype(jnp.int32)

    iw1p = jnp.pad(iw1, ((0, 0), (0, _NGP - _NG), (0, 0)))

    zp = jnp.pad(z.astype(jnp.int32), (0, padn))
    h = _sc_embed(emb.astype(jnp.float32), zp)

    xj = _mm(h, cw1[0])
    for i in range(6):
        agg = _banded(tlo, tcnt, nodef, nodet, xj, iw1p[i], ib1[i], iw2[i],
                      ib2[i])
        h, xj = _update(h, agg, cw2[i], cb2[i], lw[i], lb[i],
                        cw1[(i + 1) % 6])

    ne = _lin1(h, lin1_w, lin1_b)
    pair = ne[:n].reshape(n // 2, 2 * ne.shape[1])
    mp = 5120
    pairp = jnp.pad(pair, ((0, mp - n // 2), (0, 0)))
    outp = _readout(pairp, m1w, m1b, m2w, m2b)
    return outp[: n // 2, 0]


# speedup vs baseline: 2.2181x; 1.0563x over previous
import functools

import jax
import jax.numpy as jnp
from jax import lax
from jax.experimental import pallas as pl
from jax.experimental.pallas import tpu as pltpu
from jax.experimental.pallas import tpu_sc as plsc

_CUTOFF = 10.0
_HID = 128
_NG = 50
_NGP = 64
_R = 64
_CT = 128
_NP = 10240
_NB = _NP // _R
_NCT = _NP // _CT
_P = _R * _CT


def _ssp(x):
    return jnp.maximum(x, 0.0) + jnp.log1p(jnp.exp(-jnp.abs(x))) - jnp.log(2.0)


def _sc_embed(emb, zp):
    info = plsc.get_sparse_core_info()
    nc, ns = info.num_cores, info.num_subcores
    nw = nc * ns
    b_per_w = _NP // nw
    d = emb.shape[1]
    mesh = plsc.VectorSubcoreMesh(core_axis_name="c", subcore_axis_name="s")

    @functools.partial(
        pl.kernel,
        mesh=mesh,
        out_type=jax.ShapeDtypeStruct((_NP, d), jnp.float32),
        scratch_types=[
            pltpu.VMEM((b_per_w,), jnp.int32),
            pltpu.VMEM((b_per_w, d), jnp.float32),
            pltpu.SemaphoreType.DMA,
        ],
    )
    def gather_kernel(table_hbm, idx_hbm, out_hbm, idx_v, rows_v, sem):
        wid = lax.axis_index("s") * nc + lax.axis_index("c")
        base = wid * b_per_w
        pltpu.sync_copy(idx_hbm.at[pl.ds(base, b_per_w)], idx_v)
        pltpu.async_copy(table_hbm.at[idx_v], rows_v, sem).wait()
        pltpu.sync_copy(rows_v, out_hbm.at[pl.ds(base, b_per_w)])

    return gather_kernel(emb, zp)


def _banded_body(qk_ref, c08_ref, tcnt_ref, nodef_ref, nodet_ref, xj_ref,
                 iw1_ref, ib1_ref, iw2_ref, ib2_ref, selc_ref, out_ref):
    b = pl.program_id(0)
    qk = qk_ref[b]
    c08 = c08_ref[b]
    tcnt = tcnt_ref[b]
    r0 = b * _R

    f32 = jnp.float32
    step = _CUTOFF / (_NG - 1)
    coeff = -0.5 / (step * step)
    gi = lax.broadcasted_iota(jnp.int32, (1, _NGP), 1).astype(f32)
    off = jnp.where(gi < _NG, gi * step, 1e6)

    rowdat = nodef_ref[pl.ds(r0, _R), :]
    br = rowdat[:, 0:1]
    nr = rowdat[:, 1:2]
    prx = rowdat[:, 2:3]
    pry = rowdat[:, 3:4]
    prz = rowdat[:, 4:5]
    ridx = r0 + lax.broadcasted_iota(jnp.int32, (_R, 1), 0)

    iw1v = iw1_ref[...]
    ib1v = ib1_ref[...]
    iw2v = iw2_ref[...]
    ib2v = ib2_ref[...]

    sel_c = selc_ref[...]

    def _flatten2(a2, b2):
        ab = jnp.concatenate([a2, b2], axis=1)
        ex = jnp.repeat(ab, _CT, axis=0)
        af = jnp.sum(ex[:, :_CT] * sel_c, axis=1, keepdims=True)
        bf = jnp.sum(ex[:, _CT:] * sel_c, axis=1, keepdims=True)
        return af, bf

    def tile_body(t, acc):
        c0 = c08 + t * _CT
        colt = nodet_ref[pl.ds(qk + t * 8, 8), :]
        xjc = xj_ref[pl.ds(c0, _CT), :]
        bc = colt[0:1, :]
        nc = colt[1:2, :]
        pcx = colt[2:3, :]
        pcy = colt[3:4, :]
        pcz = colt[4:5, :]
        cidx = c0 + lax.broadcasted_iota(jnp.int32, (1, _CT), 1)

        dot3 = prx * pcx + pry * pcy + prz * pcz
        d2 = (nr + nc) - 2.0 * dot3
        m = (d2 < _CUTOFF * _CUTOFF) & (br == bc) & (ridx != cidx)
        dx = prx - pcx
        dy = pry - pcy
        dz = prz - pcz
        ew = jnp.sqrt(dx * dx + dy * dy + dz * dz)
        cw = 0.5 * (jnp.cos(ew * (jnp.pi / _CUTOFF)) + 1.0)
        cm2 = jnp.where(m, cw, 0.0)

        ew_f, cm_f = _flatten2(ew, cm2)
        ea = jnp.exp(coeff * (ew_f - off) ** 2)
        t1 = _ssp(jnp.dot(ea, iw1v, preferred_element_type=f32) + ib1v)
        w = jnp.dot(t1, iw2v, preferred_element_type=f32) + ib2v
        v = w * cm_f
        v3 = v.reshape(_R, _CT, _HID)
        return acc + jnp.sum(v3 * xjc[None, :, :], axis=1)

    acc = lax.fori_loop(0, tcnt, tile_body, jnp.zeros((_R, _HID), f32))
    out_ref[...] = acc


def _banded(qk, c08, tcnt, nodef, nodet, xj, iw1p, ib1, iw2, ib2):
    grid_spec = pltpu.PrefetchScalarGridSpec(
        num_scalar_prefetch=3,
        grid=(_NB,),
        in_specs=[
            pl.BlockSpec((_NP, 8), lambda b, *_: (0, 0)),
            pl.BlockSpec((16 * _NCT * 8, _CT), lambda b, *_: (0, 0)),
            pl.BlockSpec((_NP, _HID), lambda b, *_: (0, 0)),
            pl.BlockSpec((_NGP, _HID), lambda b, *_: (0, 0)),
            pl.BlockSpec((1, _HID), lambda b, *_: (0, 0)),
            pl.BlockSpec((_HID, _HID), lambda b, *_: (0, 0)),
            pl.BlockSpec((1, _HID), lambda b, *_: (0, 0)),
            pl.BlockSpec((_P, _CT), lambda b, *_: (0, 0)),
        ],
        out_specs=pl.BlockSpec((_R, _HID), lambda b, *_: (b, 0)),
    )
    selc = (jnp.arange(_P, dtype=jnp.int32)[:, None] % _CT
            == jnp.arange(_CT, dtype=jnp.int32)[None, :]).astype(jnp.float32)
    return pl.pallas_call(
        _banded_body,
        grid_spec=grid_spec,
        out_shape=jax.ShapeDtypeStruct((_NP, _HID), jnp.float32),
    )(qk, c08, tcnt, nodef, nodet, xj, iw1p, ib1.reshape(1, _HID), iw2,
      ib2.reshape(1, _HID), selc)


def _mm_body(x_ref, w_ref, o_ref):
    o_ref[...] = jnp.dot(x_ref[...], w_ref[...],
                         preferred_element_type=jnp.float32)


def _mm(x, w):
    m, k = x.shape
    n = w.shape[1]
    blk = min(1024, m)
    return pl.pallas_call(
        _mm_body,
        grid=(m // blk,),
        in_specs=[pl.BlockSpec((blk, k), lambda i: (i, 0)),
                  pl.BlockSpec((k, n), lambda i: (0, 0))],
        out_specs=pl.BlockSpec((blk, n), lambda i: (i, 0)),
        out_shape=jax.ShapeDtypeStruct((m, n), jnp.float32),
    )(x, w)


def _update_body(h_ref, agg_ref, cw2_ref, cb2_ref, lw_ref, lb_ref, cw1n_ref,
                 h_out, xj_out):
    t = _ssp(jnp.dot(agg_ref[...], cw2_ref[...],
                     preferred_element_type=jnp.float32) + cb2_ref[...])
    hn = h_ref[...] + jnp.dot(
        t, lw_ref[...], preferred_element_type=jnp.float32) + lb_ref[...]
    h_out[...] = hn
    xj_out[...] = jnp.dot(hn, cw1n_ref[...],
                          preferred_element_type=jnp.float32)


def _update(h, agg, cw2, cb2, lw, lb, cw1n):
    blk = min(1024, _NP)
    return pl.pallas_call(
        _update_body,
        grid=(_NP // blk,),
        in_specs=[pl.BlockSpec((blk, _HID), lambda i: (i, 0)),
                  pl.BlockSpec((blk, _HID), lambda i: (i, 0)),
                  pl.BlockSpec((_HID, _HID), lambda i: (0, 0)),
                  pl.BlockSpec((1, _HID), lambda i: (0, 0)),
                  pl.BlockSpec((_HID, _HID), lambda i: (0, 0)),
                  pl.BlockSpec((1, _HID), lambda i: (0, 0)),
                  pl.BlockSpec((_HID, _HID), lambda i: (0, 0))],
        out_specs=[pl.BlockSpec((blk, _HID), lambda i: (i, 0)),
                   pl.BlockSpec((blk, _HID), lambda i: (i, 0))],
        out_shape=[jax.ShapeDtypeStruct((_NP, _HID), jnp.float32),
                   jax.ShapeDtypeStruct((_NP, _HID), jnp.float32)],
    )(h, agg, cw2, cb2.reshape(1, _HID), lw, lb.reshape(1, _HID), cw1n)


def _lin1_body(h_ref, w_ref, b_ref, o_ref):
    o_ref[...] = jnp.dot(h_ref[...], w_ref[...],
                         preferred_element_type=jnp.float32) + b_ref[...]


def _lin1(h, w, b):
    blk = min(1024, _NP)
    n = w.shape[1]
    return pl.pallas_call(
        _lin1_body,
        grid=(_NP // blk,),
        in_specs=[pl.BlockSpec((blk, _HID), lambda i: (i, 0)),
                  pl.BlockSpec((_HID, n), lambda i: (0, 0)),
                  pl.BlockSpec((1, n), lambda i: (0, 0))],
        out_specs=pl.BlockSpec((blk, n), lambda i: (i, 0)),
        out_shape=jax.ShapeDtypeStruct((_NP, n), jnp.float32),
    )(h, w, b.reshape(1, n))


def _readout_body(p_ref, m1w_ref, m1b_ref, m2w_ref, m2b_ref, o_ref):
    t = jax.nn.relu(jnp.dot(p_ref[...], m1w_ref[...],
                            preferred_element_type=jnp.float32) + m1b_ref[...])
    o_ref[...] = jnp.dot(t, m2w_ref[...],
                         preferred_element_type=jnp.float32) + m2b_ref[...]


def _readout(pairp, m1w, m1b, m2w, m2b):
    mp = pairp.shape[0]
    blk = 512
    return pl.pallas_call(
        _readout_body,
        grid=(mp // blk,),
        in_specs=[pl.BlockSpec((blk, _HID), lambda i: (i, 0)),
                  pl.BlockSpec((_HID, _HID), lambda i: (0, 0)),
                  pl.BlockSpec((1, _HID), lambda i: (0, 0)),
                  pl.BlockSpec((_HID, 1), lambda i: (0, 0)),
                  pl.BlockSpec((1, 1), lambda i: (0, 0))],
        out_specs=pl.BlockSpec((blk, 1), lambda i: (i, 0)),
        out_shape=jax.ShapeDtypeStruct((mp, 1), jnp.float32),
    )(pairp, m1w, m1b.reshape(1, _HID), m2w, m2b.reshape(1, 1))


def kernel(z, batch, pos, edges, emb, iw1, ib1, iw2, ib2, cw1, cw2, cb2,
           lw, lb, lin1_w, lin1_b, m1w, m1b, m2w, m2b):
    n = pos.shape[0]
    flat = edges[0].reshape(-1)
    pos_s = jnp.take(pos, flat, axis=0).astype(jnp.float32)
    nrm = (pos_s * pos_s).sum(1)
    batch_i = batch.astype(jnp.int32)

    padn = _NP - n
    batch_f = jnp.pad(batch_i, (0, padn),
                      constant_values=2 ** 24 - 1).astype(jnp.float32)
    nrm_p = jnp.pad(nrm, (0, padn))
    pos_p = jnp.pad(pos_s, ((0, padn), (0, 0)))
    gidx = jnp.arange(_NP, dtype=jnp.float32)
    zeros = jnp.zeros((_NP,), jnp.float32)
    nodef = jnp.stack([batch_f, nrm_p, pos_p[:, 0], pos_p[:, 1],
                       pos_p[:, 2], gidx, zeros, zeros], axis=1)
    sent = jnp.full((_CT, 8), 0.0, jnp.float32).at[:, 0].set(2.0 ** 24 - 1)
    nodefx = jnp.concatenate([nodef, sent], axis=0)
    views = [nodefx[8 * k: 8 * k + _NP, :] for k in range(16)]
    nodet = jnp.concatenate(
        [v.T.reshape(8, _NCT, _CT).transpose(1, 0, 2).reshape(_NCT * 8, _CT)
         for v in views], axis=0)

    row0 = jnp.arange(_NB, dtype=jnp.int32) * _R
    rlast = jnp.minimum(row0 + _R - 1, n - 1)
    bfirst = batch_i[jnp.minimum(row0, n - 1)]
    cs = jnp.searchsorted(batch_i, bfirst, side="left").astype(jnp.int32)
    ce = jnp.searchsorted(batch_i, batch_i[rlast], side="right").astype(jnp.int32)
    c08 = (cs // 8) * 8
    tcnt = jnp.where(row0 < n,
                     (ce - c08 + _CT - 1) // _CT, 0).astype(jnp.int32)
    q = c08 // _CT
    kk = (c08 % _CT) // 8
    qk = (kk * _NCT + q) * 8

    iw1p = jnp.pad(iw1, ((0, 0), (0, _NGP - _NG), (0, 0)))

    zp = jnp.pad(z.astype(jnp.int32), (0, padn))
    h = _sc_embed(emb.astype(jnp.float32), zp)

    xj = _mm(h, cw1[0])
    for i in range(6):
        agg = _banded(qk, c08, tcnt, nodef, nodet, xj, iw1p[i], ib1[i],
                      iw2[i], ib2[i])
        h, xj = _update(h, agg, cw2[i], cb2[i], lw[i], lb[i],
                        cw1[(i + 1) % 6])

    ne = _lin1(h, lin1_w, lin1_b)
    pair = ne[:n].reshape(n // 2, 2 * ne.shape[1])
    mp = 5120
    pairp = jnp.pad(pair, ((0, mp - n // 2), (0, 0)))
    outp = _readout(pairp, m1w, m1b, m2w, m2b)
    return outp[: n // 2, 0]


# speedup vs baseline: 2.9890x; 1.3476x over previous
import functools

import jax
import jax.numpy as jnp
from jax import lax
from jax.experimental import pallas as pl
from jax.experimental.pallas import tpu as pltpu
from jax.experimental.pallas import tpu_sc as plsc

_CUTOFF = 10.0
_HID = 128
_NG = 50
_NGP = 64
_R = 64
_RH = 32
_W = 64
_NP = 10240
_NB = _NP // _R
_NCW = _NP // _W
_P = _R * _W


def _ssp(x):
    return jnp.maximum(x, 0.0) + jnp.log1p(jnp.exp(-jnp.abs(x))) - jnp.log(2.0)


def _sc_embed(emb, zp):
    info = plsc.get_sparse_core_info()
    nc, ns = info.num_cores, info.num_subcores
    nw = nc * ns
    b_per_w = _NP // nw
    d = emb.shape[1]
    mesh = plsc.VectorSubcoreMesh(core_axis_name="c", subcore_axis_name="s")

    @functools.partial(
        pl.kernel,
        mesh=mesh,
        out_type=jax.ShapeDtypeStruct((_NP, d), jnp.float32),
        scratch_types=[
            pltpu.VMEM((b_per_w,), jnp.int32),
            pltpu.VMEM((b_per_w, d), jnp.float32),
            pltpu.SemaphoreType.DMA,
        ],
    )
    def gather_kernel(table_hbm, idx_hbm, out_hbm, idx_v, rows_v, sem):
        wid = lax.axis_index("s") * nc + lax.axis_index("c")
        base = wid * b_per_w
        pltpu.sync_copy(idx_hbm.at[pl.ds(base, b_per_w)], idx_v)
        pltpu.async_copy(table_hbm.at[idx_v], rows_v, sem).wait()
        pltpu.sync_copy(rows_v, out_hbm.at[pl.ds(base, b_per_w)])

    return gather_kernel(emb, zp)


def _banded_body(qka_ref, c0a_ref, qkb_ref, c0b_ref, tcnt_ref, nodef_ref,
                 nodet_ref, xj_ref, iw1_ref, ib1_ref, iw2_ref, ib2_ref,
                 selc_ref, out_ref):
    b = pl.program_id(0)
    qka = qka_ref[b]
    c0a = c0a_ref[b]
    qkb = qkb_ref[b]
    c0b = c0b_ref[b]
    tcnt = tcnt_ref[b]
    r0 = b * _R
    lima = (_NP - _W - c0a) // _W
    limb = (_NP - _W - c0b) // _W

    f32 = jnp.float32
    step = _CUTOFF / (_NG - 1)
    coeff = -0.5 / (step * step)
    gi = lax.broadcasted_iota(jnp.int32, (1, _NGP), 1).astype(f32)
    off = jnp.where(gi < _NG, gi * step, 1e6)

    rowdat = nodef_ref[pl.ds(r0, _R), :]
    br = rowdat[:, 0:1]
    nr = rowdat[:, 1:2]
    prx = rowdat[:, 2:3]
    pry = rowdat[:, 3:4]
    prz = rowdat[:, 4:5]
    ridx = r0 + lax.broadcasted_iota(jnp.int32, (_R, 1), 0)

    iw1v = iw1_ref[...]
    ib1v = ib1_ref[...]
    iw2v = iw2_ref[...]
    ib2v = ib2_ref[...]

    sel_c = selc_ref[...]

    def _flatten2(a2, b2):
        ab = jnp.concatenate([a2, b2], axis=1)
        ex = jnp.repeat(ab, _W, axis=0)
        af = jnp.sum(ex[:, :_W] * sel_c, axis=1, keepdims=True)
        bf = jnp.sum(ex[:, _W:] * sel_c, axis=1, keepdims=True)
        return af, bf

    def _asm(a2, b2):
        return jnp.concatenate([jnp.broadcast_to(a2, (_RH, _W)),
                                jnp.broadcast_to(b2, (_RH, _W))], axis=0)

    def tile_body(t, acc):
        ta = jnp.minimum(t, lima)
        tb = jnp.minimum(t, limb)
        c0at = c0a + ta * _W
        c0bt = c0b + tb * _W
        cola = nodet_ref[pl.ds(qka + 8 * ta, 8), :]
        colb = nodet_ref[pl.ds(qkb + 8 * tb, 8), :]
        xja = xj_ref[pl.ds(c0at, _W), :]
        xjb = xj_ref[pl.ds(c0bt, _W), :]
        wi = lax.broadcasted_iota(jnp.int32, (1, _W), 1)

        bc = _asm(cola[0:1, :], colb[0:1, :])
        nc = _asm(cola[1:2, :], colb[1:2, :])
        pcx = _asm(cola[2:3, :], colb[2:3, :])
        pcy = _asm(cola[3:4, :], colb[3:4, :])
        pcz = _asm(cola[4:5, :], colb[4:5, :])
        cidx = _asm(c0at + wi, c0bt + wi)

        dot3 = prx * pcx + pry * pcy + prz * pcz
        d2 = (nr + nc) - 2.0 * dot3
        m = (d2 < _CUTOFF * _CUTOFF) & (br == bc) & (ridx != cidx)
        dx = prx - pcx
        dy = pry - pcy
        dz = prz - pcz
        ew = jnp.sqrt(dx * dx + dy * dy + dz * dz)
        cw = 0.5 * (jnp.cos(ew * (jnp.pi / _CUTOFF)) + 1.0)
        cm2 = jnp.where(m, cw, 0.0)

        ew_f, cm_f = _flatten2(ew, cm2)
        ea = jnp.exp(coeff * (ew_f - off) ** 2)
        t1 = _ssp(jnp.dot(ea, iw1v, preferred_element_type=f32) + ib1v)
        w = jnp.dot(t1, iw2v, preferred_element_type=f32) + ib2v
        v = w * cm_f
        va = v[: _RH * _W].reshape(_RH, _W, _HID)
        vb = v[_RH * _W:].reshape(_RH, _W, _HID)
        ca = jnp.sum(va * xja[None, :, :], axis=1)
        cb = jnp.sum(vb * xjb[None, :, :], axis=1)
        return acc + jnp.concatenate([ca, cb], axis=0)

    acc = lax.fori_loop(0, tcnt, tile_body, jnp.zeros((_R, _HID), f32))
    out_ref[...] = acc


def _banded(qka, c0a, qkb, c0b, tcnt, nodef, nodet, xj, iw1p, ib1, iw2, ib2):
    grid_spec = pltpu.PrefetchScalarGridSpec(
        num_scalar_prefetch=5,
        grid=(_NB,),
        in_specs=[
            pl.BlockSpec((_NP, 8), lambda b, *_: (0, 0)),
            pl.BlockSpec((8 * _NCW * 8, _W), lambda b, *_: (0, 0)),
            pl.BlockSpec((_NP, _HID), lambda b, *_: (0, 0)),
            pl.BlockSpec((_NGP, _HID), lambda b, *_: (0, 0)),
            pl.BlockSpec((1, _HID), lambda b, *_: (0, 0)),
            pl.BlockSpec((_HID, _HID), lambda b, *_: (0, 0)),
            pl.BlockSpec((1, _HID), lambda b, *_: (0, 0)),
            pl.BlockSpec((_P, _W), lambda b, *_: (0, 0)),
        ],
        out_specs=pl.BlockSpec((_R, _HID), lambda b, *_: (b, 0)),
    )
    selc = (jnp.arange(_P, dtype=jnp.int32)[:, None] % _W
            == jnp.arange(_W, dtype=jnp.int32)[None, :]).astype(jnp.float32)
    return pl.pallas_call(
        _banded_body,
        grid_spec=grid_spec,
        out_shape=jax.ShapeDtypeStruct((_NP, _HID), jnp.float32),
    )(qka, c0a, qkb, c0b, tcnt, nodef, nodet, xj, iw1p,
      ib1.reshape(1, _HID), iw2, ib2.reshape(1, _HID), selc)


def _mm_body(x_ref, w_ref, o_ref):
    o_ref[...] = jnp.dot(x_ref[...], w_ref[...],
                         preferred_element_type=jnp.float32)


def _mm(x, w):
    m, k = x.shape
    n = w.shape[1]
    blk = min(1024, m)
    return pl.pallas_call(
        _mm_body,
        grid=(m // blk,),
        in_specs=[pl.BlockSpec((blk, k), lambda i: (i, 0)),
                  pl.BlockSpec((k, n), lambda i: (0, 0))],
        out_specs=pl.BlockSpec((blk, n), lambda i: (i, 0)),
        out_shape=jax.ShapeDtypeStruct((m, n), jnp.float32),
    )(x, w)


def _update_body(h_ref, agg_ref, cw2_ref, cb2_ref, lw_ref, lb_ref, cw1n_ref,
                 h_out, xj_out):
    t = _ssp(jnp.dot(agg_ref[...], cw2_ref[...],
                     preferred_element_type=jnp.float32) + cb2_ref[...])
    hn = h_ref[...] + jnp.dot(
        t, lw_ref[...], preferred_element_type=jnp.float32) + lb_ref[...]
    h_out[...] = hn
    xj_out[...] = jnp.dot(hn, cw1n_ref[...],
                          preferred_element_type=jnp.float32)


def _update(h, agg, cw2, cb2, lw, lb, cw1n):
    blk = min(1024, _NP)
    return pl.pallas_call(
        _update_body,
        grid=(_NP // blk,),
        in_specs=[pl.BlockSpec((blk, _HID), lambda i: (i, 0)),
                  pl.BlockSpec((blk, _HID), lambda i: (i, 0)),
                  pl.BlockSpec((_HID, _HID), lambda i: (0, 0)),
                  pl.BlockSpec((1, _HID), lambda i: (0, 0)),
                  pl.BlockSpec((_HID, _HID), lambda i: (0, 0)),
                  pl.BlockSpec((1, _HID), lambda i: (0, 0)),
                  pl.BlockSpec((_HID, _HID), lambda i: (0, 0))],
        out_specs=[pl.BlockSpec((blk, _HID), lambda i: (i, 0)),
                   pl.BlockSpec((blk, _HID), lambda i: (i, 0))],
        out_shape=[jax.ShapeDtypeStruct((_NP, _HID), jnp.float32),
                   jax.ShapeDtypeStruct((_NP, _HID), jnp.float32)],
    )(h, agg, cw2, cb2.reshape(1, _HID), lw, lb.reshape(1, _HID), cw1n)


def _lin1_body(h_ref, w_ref, b_ref, o_ref):
    o_ref[...] = jnp.dot(h_ref[...], w_ref[...],
                         preferred_element_type=jnp.float32) + b_ref[...]


def _lin1(h, w, b):
    blk = min(1024, _NP)
    n = w.shape[1]
    return pl.pallas_call(
        _lin1_body,
        grid=(_NP // blk,),
        in_specs=[pl.BlockSpec((blk, _HID), lambda i: (i, 0)),
                  pl.BlockSpec((_HID, n), lambda i: (0, 0)),
                  pl.BlockSpec((1, n), lambda i: (0, 0))],
        out_specs=pl.BlockSpec((blk, n), lambda i: (i, 0)),
        out_shape=jax.ShapeDtypeStruct((_NP, n), jnp.float32),
    )(h, w, b.reshape(1, n))


def _readout_body(p_ref, m1w_ref, m1b_ref, m2w_ref, m2b_ref, o_ref):
    t = jax.nn.relu(jnp.dot(p_ref[...], m1w_ref[...],
                            preferred_element_type=jnp.float32) + m1b_ref[...])
    o_ref[...] = jnp.dot(t, m2w_ref[...],
                         preferred_element_type=jnp.float32) + m2b_ref[...]


def _readout(pairp, m1w, m1b, m2w, m2b):
    mp = pairp.shape[0]
    blk = 512
    return pl.pallas_call(
        _readout_body,
        grid=(mp // blk,),
        in_specs=[pl.BlockSpec((blk, _HID), lambda i: (i, 0)),
                  pl.BlockSpec((_HID, _HID), lambda i: (0, 0)),
                  pl.BlockSpec((1, _HID), lambda i: (0, 0)),
                  pl.BlockSpec((_HID, 1), lambda i: (0, 0)),
                  pl.BlockSpec((1, 1), lambda i: (0, 0))],
        out_specs=pl.BlockSpec((blk, 1), lambda i: (i, 0)),
        out_shape=jax.ShapeDtypeStruct((mp, 1), jnp.float32),
    )(pairp, m1w, m1b.reshape(1, _HID), m2w, m2b.reshape(1, 1))


def kernel(z, batch, pos, edges, emb, iw1, ib1, iw2, ib2, cw1, cw2, cb2,
           lw, lb, lin1_w, lin1_b, m1w, m1b, m2w, m2b):
    n = pos.shape[0]
    flat = edges[0].reshape(-1)
    pos_s = jnp.take(pos, flat, axis=0).astype(jnp.float32)
    nrm = (pos_s * pos_s).sum(1)
    batch_i = batch.astype(jnp.int32)

    padn = _NP - n
    batch_f = jnp.pad(batch_i, (0, padn),
                      constant_values=2 ** 24 - 1).astype(jnp.float32)
    nrm_p = jnp.pad(nrm, (0, padn))
    pos_p = jnp.pad(pos_s, ((0, padn), (0, 0)))
    gidx = jnp.arange(_NP, dtype=jnp.float32)
    zeros = jnp.zeros((_NP,), jnp.float32)
    nodef = jnp.stack([batch_f, nrm_p, pos_p[:, 0], pos_p[:, 1],
                       pos_p[:, 2], gidx, zeros, zeros], axis=1)
    sent = jnp.full((_W, 8), 0.0, jnp.float32).at[:, 0].set(2.0 ** 24 - 1)
    nodefx = jnp.concatenate([nodef, sent], axis=0)
    views = [nodefx[8 * k: 8 * k + _NP, :] for k in range(8)]
    nodet = jnp.concatenate(
        [v.T.reshape(8, _NCW, _W).transpose(1, 0, 2).reshape(_NCW * 8, _W)
         for v in views], axis=0)

    row0h = jnp.arange(2 * _NB, dtype=jnp.int32) * _RH
    rlasth = jnp.minimum(row0h + _RH - 1, n - 1)
    bfh = batch_i[jnp.minimum(row0h, n - 1)]
    csh = jnp.searchsorted(batch_i, bfh, side="left").astype(jnp.int32)
    ceh = jnp.searchsorted(batch_i, batch_i[rlasth],
                           side="right").astype(jnp.int32)
    c0h = (csh // 8) * 8
    tch = jnp.where(row0h < n,
                    (ceh - c0h + _W - 1) // _W, 0).astype(jnp.int32)
    qh = c0h // _W
    kh = (c0h % _W) // 8
    qkh = (kh * _NCW + qh) * 8
    qka, qkb = qkh[0::2], qkh[1::2]
    c0a, c0b = c0h[0::2], c0h[1::2]
    tcnt = jnp.maximum(tch[0::2], tch[1::2])

    iw1p = jnp.pad(iw1, ((0, 0), (0, _NGP - _NG), (0, 0)))

    zp = jnp.pad(z.astype(jnp.int32), (0, padn))
    h = _sc_embed(emb.astype(jnp.float32), zp)

    xj = _mm(h, cw1[0])
    for i in range(6):
        agg = _banded(qka, c0a, qkb, c0b, tcnt, nodef, nodet, xj, iw1p[i],
                      ib1[i], iw2[i], ib2[i])
        h, xj = _update(h, agg, cw2[i], cb2[i], lw[i], lb[i],
                        cw1[(i + 1) % 6])

    ne = _lin1(h, lin1_w, lin1_b)
    pair = ne[:n].reshape(n // 2, 2 * ne.shape[1])
    mp = 5120
    pairp = jnp.pad(pair, ((0, mp - n // 2), (0, 0)))
    outp = _readout(pairp, m1w, m1b, m2w, m2b)
    return outp[: n // 2, 0]
